# baseline (speedup 1.0000x reference)
import functools

import jax
import jax.numpy as jnp
from jax import lax
from jax.experimental import pallas as pl
from jax.experimental.pallas import tpu as pltpu
from jax.experimental.pallas import tpu_sc as plsc

N_ROWS = 320000
D_FEAT = 128
N_SEG = 10000
SEG_PAD = 10240

NC = 2
NS = 16
NW = NC * NS
ROWS_PER_TILE = N_ROWS // NW
CHUNK = 64
NBUF = 3
NQUAD = 52
NMAIN = NBUF * NQUAD
TAIL = ROWS_PER_TILE - NMAIN * CHUNK
SEG_PER_TILE = SEG_PAD // NS

_mesh = plsc.VectorSubcoreMesh(core_axis_name="c", subcore_axis_name="s")


@functools.partial(
    pl.kernel,
    mesh=_mesh,
    out_type=jax.ShapeDtypeStruct((NC, SEG_PAD, D_FEAT), jnp.float32),
    scratch_types=[
        pltpu.VMEM((NMAIN, CHUNK), jnp.int32),
        pltpu.VMEM((1, TAIL), jnp.int32),
        pltpu.VMEM((CHUNK, D_FEAT), jnp.float32),
        pltpu.VMEM((CHUNK, D_FEAT), jnp.float32),
        pltpu.VMEM((CHUNK, D_FEAT), jnp.float32),
        pltpu.VMEM((TAIL, D_FEAT), jnp.float32),
        pltpu.VMEM_SHARED((SEG_PAD, D_FEAT), jnp.float32),
        pltpu.SemaphoreType.DMA,
        pltpu.SemaphoreType.DMA,
        pltpu.SemaphoreType.DMA,
        pltpu.SemaphoreType.DMA,
        pltpu.SemaphoreType.DMA,
        pltpu.SemaphoreType.DMA,
    ],
)
def _segment_sum_sc(data_hbm, seg_hbm, segt_hbm, zeros_hbm, out_hbm,
                    ids_v, ids_t, b0, b1, b2, bt, acc,
                    l0, l1, l2, s0, s1, s2):
    c = lax.axis_index("c")
    s = lax.axis_index("s")
    wid = c * NS + s
    base0 = wid * ROWS_PER_TILE
    bufs = (b0, b1, b2)
    lsems = (l0, l1, l2)
    ssems = (s0, s1, s2)

    pltpu.sync_copy(zeros_hbm, acc.at[pl.ds(s * SEG_PER_TILE, SEG_PER_TILE)])
    plsc.subcore_barrier()

    pltpu.sync_copy(seg_hbm.at[wid], ids_v)
    pltpu.sync_copy(segt_hbm.at[wid], ids_t)

    def load(j, b):
        pltpu.async_copy(data_hbm.at[pl.ds(base0 + j * CHUNK, CHUNK)],
                         bufs[b], lsems[b])

    def wait_load(j, b):
        pltpu.make_async_copy(data_hbm.at[pl.ds(base0 + j * CHUNK, CHUNK)],
                              bufs[b], lsems[b]).wait()

    def wait_scatter(j, b):
        pltpu.make_async_copy(bufs[b], acc.at[ids_v.at[j]], ssems[b]).wait()

    for b in range(NBUF):
        load(b, b)

    def body(i, carry):
        j0 = i * NBUF
        for b in range(NBUF):
            wait_load(j0 + b, b)
            pltpu.async_copy(bufs[b], acc.at[ids_v.at[j0 + b]], ssems[b],
                             add=True)
        for b in range(NBUF):
            wait_scatter(j0 + b, b)

            @pl.when(i < NQUAD - 1)
            def _():
                load(j0 + NBUF + b, b)

        return carry

    lax.fori_loop(0, NQUAD, body, 0)

    pltpu.sync_copy(data_hbm.at[pl.ds(base0 + NMAIN * CHUNK, TAIL)], bt)
    pltpu.sync_copy(bt, acc.at[ids_t.at[0]], add=True)

    plsc.subcore_barrier()

    pltpu.sync_copy(
        acc.at[pl.ds(s * SEG_PER_TILE, SEG_PER_TILE)],
        out_hbm.at[c, pl.ds(s * SEG_PER_TILE, SEG_PER_TILE)],
    )


def _combine_body(p_ref, o_ref):
    o_ref[...] = p_ref[0] + p_ref[1]


def _combine(partials):
    nblk = 10
    rows = N_SEG // nblk
    return pl.pallas_call(
        _combine_body,
        out_shape=jax.ShapeDtypeStruct((N_SEG, D_FEAT), jnp.float32),
        grid=(nblk,),
        in_specs=[pl.BlockSpec((NC, rows, D_FEAT), lambda i: (0, i, 0))],
        out_specs=pl.BlockSpec((rows, D_FEAT), lambda i: (i, 0)),
    )(partials)


def kernel(data, segment_ids):
    seg = segment_ids.astype(jnp.int32).reshape(NW, ROWS_PER_TILE)
    seg_main = seg[:, : NMAIN * CHUNK].reshape(NW, NMAIN, CHUNK)
    seg_tail = seg[:, NMAIN * CHUNK :].reshape(NW, 1, TAIL)
    zeros = jnp.zeros((SEG_PER_TILE, D_FEAT), jnp.float32)
    partials = _segment_sum_sc(data, seg_main, seg_tail, zeros)
    return _combine(partials)


# speedup vs baseline: 1.1143x; 1.1143x over previous
import functools

import jax
import jax.numpy as jnp
from jax import lax
from jax.experimental import pallas as pl
from jax.experimental.pallas import tpu as pltpu
from jax.experimental.pallas import tpu_sc as plsc

N_ROWS = 320000
D_FEAT = 128
N_SEG = 10000
SEG_PAD = 10240

NC = 2
NS = 16
NW = NC * NS
ROWS_PER_TILE = N_ROWS // NW
CHUNK = 128
NPAIR = 39
NMAIN = 2 * NPAIR
TAIL = ROWS_PER_TILE - NMAIN * CHUNK
SEG_PER_TILE = SEG_PAD // NS

_mesh = plsc.VectorSubcoreMesh(core_axis_name="c", subcore_axis_name="s")


@functools.partial(
    pl.kernel,
    mesh=_mesh,
    out_type=jax.ShapeDtypeStruct((NC, SEG_PAD, D_FEAT), jnp.float32),
    scratch_types=[
        pltpu.VMEM((NMAIN, CHUNK), jnp.int32),
        pltpu.VMEM((1, TAIL), jnp.int32),
        pltpu.VMEM((CHUNK, D_FEAT), jnp.float32),
        pltpu.VMEM((CHUNK, D_FEAT), jnp.float32),
        pltpu.VMEM((TAIL, D_FEAT), jnp.float32),
        pltpu.VMEM_SHARED((SEG_PAD, D_FEAT), jnp.float32),
        pltpu.SemaphoreType.DMA,
        pltpu.SemaphoreType.DMA,
    ],
)
def _segment_sum_sc(data_hbm, seg_hbm, segt_hbm, zeros_hbm, out_hbm,
                    ids_v, ids_t, b0, b1, bt, acc, l0, l1):
    c = lax.axis_index("c")
    s = lax.axis_index("s")
    wid = c * NS + s
    base0 = wid * ROWS_PER_TILE

    pltpu.sync_copy(zeros_hbm, acc.at[pl.ds(s * SEG_PER_TILE, SEG_PER_TILE)])
    plsc.subcore_barrier()

    pltpu.sync_copy(seg_hbm.at[wid], ids_v)
    pltpu.sync_copy(segt_hbm.at[wid], ids_t)

    def load(j, buf, sem):
        pltpu.async_copy(data_hbm.at[pl.ds(base0 + j * CHUNK, CHUNK)],
                         buf, sem)

    def wait_load(j, buf, sem):
        pltpu.make_async_copy(data_hbm.at[pl.ds(base0 + j * CHUNK, CHUNK)],
                              buf, sem).wait()

    load(0, b0, l0)

    def body(i, carry):
        ch0 = 2 * i
        load(ch0 + 1, b1, l1)
        wait_load(ch0, b0, l0)
        pltpu.sync_copy(b0, acc.at[ids_v.at[ch0]], add=True)

        @pl.when(i < NPAIR - 1)
        def _():
            load(ch0 + 2, b0, l0)

        wait_load(ch0 + 1, b1, l1)
        pltpu.sync_copy(b1, acc.at[ids_v.at[ch0 + 1]], add=True)
        return carry

    lax.fori_loop(0, NPAIR, body, 0)

    pltpu.sync_copy(data_hbm.at[pl.ds(base0 + NMAIN * CHUNK, TAIL)], bt)
    pltpu.sync_copy(bt, acc.at[ids_t.at[0]], add=True)

    plsc.subcore_barrier()

    pltpu.sync_copy(
        acc.at[pl.ds(s * SEG_PER_TILE, SEG_PER_TILE)],
        out_hbm.at[c, pl.ds(s * SEG_PER_TILE, SEG_PER_TILE)],
    )


def _combine_body(p_ref, o_ref):
    o_ref[...] = p_ref[0] + p_ref[1]


def _combine(partials):
    nblk = 10
    rows = N_SEG // nblk
    return pl.pallas_call(
        _combine_body,
        out_shape=jax.ShapeDtypeStruct((N_SEG, D_FEAT), jnp.float32),
        grid=(nblk,),
        in_specs=[pl.BlockSpec((NC, rows, D_FEAT), lambda i: (0, i, 0))],
        out_specs=pl.BlockSpec((rows, D_FEAT), lambda i: (i, 0)),
    )(partials)


def kernel(data, segment_ids):
    seg = segment_ids.astype(jnp.int32).reshape(NW, ROWS_PER_TILE)
    seg_main = seg[:, : NMAIN * CHUNK].reshape(NW, NMAIN, CHUNK)
    seg_tail = seg[:, NMAIN * CHUNK :].reshape(NW, 1, TAIL)
    zeros = jnp.zeros((SEG_PER_TILE, D_FEAT), jnp.float32)
    partials = _segment_sum_sc(data, seg_main, seg_tail, zeros)
    return _combine(partials)
